# speedup vs baseline: 14.5294x; 14.5294x over previous
import functools

import jax
import jax.numpy as jnp
from jax import lax
from jax.experimental import pallas as pl
from jax.experimental.pallas import tpu as pltpu
from jax.experimental.pallas import tpu_sc as plsc

N = 10000
F = 128
E = 320000
NC = 2
NS = 16
NW = NC * NS
K = 128
NP = 10240
RPW = NP // NW
NB = -(-E // (NW * K))
EP = NW * NB * K
JUNK = NP - 1
BN_S = 1.0 / (1.0 + 1e-5) ** 0.5

_mesh = plsc.VectorSubcoreMesh(core_axis_name="c", subcore_axis_name="s")


DG = 16


def _deg_sc(dst_r, zrows, ones):
    @functools.partial(
        pl.kernel,
        out_type=jax.ShapeDtypeStruct((NC, NP, DG), jnp.float32),
        mesh=_mesh,
        scratch_types=[
            pltpu.VMEM_SHARED((NP, DG), jnp.float32),
            pltpu.VMEM((K,), jnp.int32),
            pltpu.VMEM((K, DG), jnp.float32),
        ],
    )
    def run(dst_hbm, z_hbm, ones_hbm, out_hbm, acc_sp, idx_v, ones_v):
        c = lax.axis_index("c")
        s = lax.axis_index("s")
        pltpu.sync_copy(z_hbm, acc_sp.at[pl.ds(s * RPW, RPW)])
        pltpu.sync_copy(ones_hbm, ones_v)
        plsc.subcore_barrier()

        @pl.loop(0, NB)
        def _(j):
            pltpu.sync_copy(dst_hbm.at[c, s, j], idx_v)
            pltpu.sync_copy(ones_v, acc_sp.at[idx_v], add=True)

        plsc.subcore_barrier()
        pltpu.sync_copy(acc_sp.at[pl.ds(s * RPW, RPW)],
                        out_hbm.at[c, pl.ds(s * RPW, RPW)])

    return run(dst_r, zrows, ones)


def _agg_sc(xws, src_r, dst_r, zrows):
    @functools.partial(
        pl.kernel,
        out_type=jax.ShapeDtypeStruct((NC, NP, F), jnp.float32),
        mesh=_mesh,
        scratch_types=[
            pltpu.VMEM_SHARED((NP, F), jnp.float32),
            pltpu.VMEM((K,), jnp.int32),
            pltpu.VMEM((K,), jnp.int32),
            pltpu.VMEM((K, F), jnp.float32),
        ],
    )
    def run(xws_hbm, src_hbm, dst_hbm, z_hbm, out_hbm, acc_sp, sidx_v, didx_v,
            rows_v):
        c = lax.axis_index("c")
        s = lax.axis_index("s")
        pltpu.sync_copy(z_hbm, acc_sp.at[pl.ds(s * RPW, RPW)])
        plsc.subcore_barrier()

        @pl.loop(0, NB)
        def _(j):
            pltpu.sync_copy(src_hbm.at[c, s, j], sidx_v)
            pltpu.sync_copy(dst_hbm.at[c, s, j], didx_v)
            pltpu.sync_copy(xws_hbm.at[sidx_v], rows_v)
            pltpu.sync_copy(rows_v, acc_sp.at[didx_v], add=True)

        plsc.subcore_barrier()
        pltpu.sync_copy(acc_sp.at[pl.ds(s * RPW, RPW)],
                        out_hbm.at[c, pl.ds(s * RPW, RPW)])

    return run(xws, src_r, dst_r, zrows)


BM = 1280


def _xws_tc_body(x_ref, w_ref, deg_ref, o_ref):
    xw = lax.dot_general(x_ref[...], w_ref[...], (((1,), (0,)), ((), ())),
                         precision=lax.Precision.HIGHEST,
                         preferred_element_type=jnp.float32)
    deg = deg_ref[0][:, 0:1] + deg_ref[1][:, 0:1] + 1.0
    o_ref[...] = xw * lax.rsqrt(deg)


def _xws_tc(xp, W, deg2):
    return pl.pallas_call(
        _xws_tc_body,
        grid=(NP // BM,),
        in_specs=[
            pl.BlockSpec((BM, F), lambda i: (i, 0)),
            pl.BlockSpec((F, F), lambda i: (0, 0)),
            pl.BlockSpec((NC, BM, DG), lambda i: (0, i, 0)),
        ],
        out_specs=pl.BlockSpec((BM, F), lambda i: (i, 0)),
        out_shape=jax.ShapeDtypeStruct((NP, F), jnp.float32),
    )(xp, W, deg2)


def _fin_tc_body(agg_ref, xws_ref, deg_ref, b_ref, g_ref, bt_ref, o_ref):
    deg = deg_ref[0][:, 0:1] + deg_ref[1][:, 0:1] + 1.0
    dinv = lax.rsqrt(deg)
    tot = (agg_ref[0] + agg_ref[1] + xws_ref[...]) * dinv
    y = (tot + b_ref[...]) * BN_S * g_ref[...] + bt_ref[...]
    o_ref[...] = jnp.maximum(y, 0.0)


def _fin_tc(agg, xws, deg2, b, gamma, beta):
    vec = pl.BlockSpec((1, F), lambda i: (0, 0))
    return pl.pallas_call(
        _fin_tc_body,
        grid=(NP // BM,),
        in_specs=[
            pl.BlockSpec((NC, BM, F), lambda i: (0, i, 0)),
            pl.BlockSpec((BM, F), lambda i: (i, 0)),
            pl.BlockSpec((NC, BM, DG), lambda i: (0, i, 0)),
            vec, vec, vec,
        ],
        out_specs=pl.BlockSpec((BM, F), lambda i: (i, 0)),
        out_shape=jax.ShapeDtypeStruct((NP, F), jnp.float32),
    )(agg, xws, deg2, b.reshape(1, F), gamma.reshape(1, F), beta.reshape(1, F))


def kernel(x, edge_index, W, b, gamma, beta):
    ei = edge_index.astype(jnp.int32)
    pad = jnp.full((EP - E,), JUNK, dtype=jnp.int32)
    src_r = jnp.concatenate([ei[0], pad]).reshape(NC, NS, NB, K)
    dst_r = jnp.concatenate([ei[1], pad]).reshape(NC, NS, NB, K)
    xp = jnp.pad(x, ((0, NP - N), (0, 0)))

    zdeg = jnp.zeros((RPW, DG), jnp.float32)
    ones = jnp.ones((K, DG), jnp.float32)
    zrows = jnp.zeros((RPW, F), jnp.float32)

    deg2 = _deg_sc(dst_r, zdeg, ones)
    xws = _xws_tc(xp, W, deg2)
    agg = _agg_sc(xws, src_r, dst_r, zrows)
    out = _fin_tc(agg, xws, deg2, b, gamma, beta)
    return out[:N]
